# Initial kernel scaffold; baseline (speedup 1.0000x reference)
import jax
import jax.numpy as jnp
from jax.experimental import pallas as pl


def kernel(x, embed, W_out, rand_times, perm_rand, subset_rand1, subset_rand2, random_tokens):
    raise NotImplementedError("write your pallas kernel here")


# speedup vs baseline: 1.7704x; 1.7704x over previous
import functools

import jax
import jax.numpy as jnp
import numpy as np
from jax import lax
from jax.experimental import pallas as pl
from jax.experimental.pallas import tpu as pltpu
from jax.experimental.pallas import tpu_sc as plsc

B, N, D, V = 2, 2048, 768, 8192
MASK_ID = V
SUB, LANE = 16, 128
NO_REPLACE_PROB = np.float32(0.15)
RANDOM_TOKEN_PROB_EFF = np.float32(0.05 * (1.0 - 0.15))

TBLK = 256
VBLK = 2048
NTB, NVB = (B * N) // TBLK, V // VBLK


def _bitonic_argsort(v, ilin, li, si):
    idx = ilin
    for klog in range(1, 12):
        kbit = 1 << klog
        asc = (ilin & kbit) == 0
        for jlog in reversed(range(klog)):
            d = 1 << jlog
            if d < LANE:
                ax, sh = 1, d
                lower = (li & d) == 0
            else:
                ax, sh = 0, d // LANE
                lower = (si & sh) == 0
            pv = jnp.where(lower, jnp.roll(v, -sh, axis=ax), jnp.roll(v, sh, axis=ax))
            pidx = jnp.where(lower, jnp.roll(idx, -sh, axis=ax), jnp.roll(idx, sh, axis=ax))
            gtp = (v > pv) | ((v == pv) & (idx > pidx))
            take = gtp == (asc == lower)
            v = jnp.where(take, pv, v)
            idx = jnp.where(take, pidx, idx)
    return idx


def _mask_kernel(rt_ref, x_ref, perm_ref, s1_ref, s2_ref, rtok_ref,
                 mask_ref, ids_ref):
    li = lax.broadcasted_iota(jnp.int32, (SUB, LANE), 1)
    si = lax.broadcasted_iota(jnp.int32, (SUB, LANE), 0)
    ilin = si * LANE + li

    sidx = _bitonic_argsort(perm_ref[0], ilin, li, si)
    rt = rt_ref[0, 0, 0]
    numtok = jnp.maximum((1.0 - rt) * np.float32(N), np.float32(1.0))
    mask = sidx.astype(jnp.float32) < numtok

    msum = jnp.sum(mask.astype(jnp.float32))
    p1 = np.float32(N) - msum
    nm1 = jnp.maximum(msum * NO_REPLACE_PROB, np.float32(0.0))
    a2 = jnp.where(mask, s1_ref[0], np.float32(-1.0))
    sidx2 = _bitonic_argsort(a2, ilin, li, si)
    norep = mask & ((sidx2.astype(jnp.float32) - p1) < nm1)
    rep = mask & (~norep)

    rsum = jnp.sum(rep.astype(jnp.float32))
    p2 = np.float32(N) - rsum
    nm2 = jnp.maximum(rsum * RANDOM_TOKEN_PROB_EFF, np.float32(0.0))
    a3 = jnp.where(rep, s2_ref[0], np.float32(-1.0))
    sidx3 = _bitonic_argsort(a3, ilin, li, si)
    rndm = rep & ((sidx3.astype(jnp.float32) - p2) < nm2)
    rep_final = rep & (~rndm)

    x2 = jnp.where(rndm, rtok_ref[0], x_ref[0])
    ids_ref[0] = jnp.where(rep_final, MASK_ID, x2)
    mask_ref[0] = mask.astype(jnp.float32)


def _compute_masks(rt2, xr, pr, s1r, s2r, rtokr):
    blk = pl.BlockSpec((1, SUB, LANE), lambda b: (b, 0, 0))
    return pl.pallas_call(
        _mask_kernel,
        grid=(B,),
        in_specs=[pl.BlockSpec((1, 1, 1), lambda b: (b, 0, 0), memory_space=pltpu.SMEM),
                  blk, blk, blk, blk, blk],
        out_specs=[blk, blk],
        out_shape=[
            jax.ShapeDtypeStruct((B, SUB, LANE), jnp.float32),
            jax.ShapeDtypeStruct((B, SUB, LANE), jnp.int32),
        ],
    )(rt2, xr, pr, s1r, s2r, rtokr)


_SC_CORES, _SC_SUBCORES = 2, 16
_NW = _SC_CORES * _SC_SUBCORES
_BPW = (B * N) // _NW


def _gather_body(table_hbm, idx_hbm, out_hbm, idx_v, rows_v, sem):
    wid = lax.axis_index("s") * _SC_CORES + lax.axis_index("c")
    base = wid * _BPW
    pltpu.sync_copy(idx_hbm.at[pl.ds(base, _BPW)], idx_v)
    pltpu.async_copy(table_hbm.at[idx_v], rows_v, sem).wait()
    pltpu.sync_copy(rows_v, out_hbm.at[pl.ds(base, _BPW)])


def _gather_rows(embed, ids_flat):
    mesh = plsc.VectorSubcoreMesh(core_axis_name="c", subcore_axis_name="s")
    k = pl.kernel(
        _gather_body,
        out_type=jax.ShapeDtypeStruct((B * N, D), jnp.float32),
        mesh=mesh,
        scratch_types=[
            pltpu.VMEM((_BPW,), jnp.int32),
            pltpu.VMEM((_BPW, D), jnp.float32),
            pltpu.SemaphoreType.DMA,
        ],
    )
    return k(embed, ids_flat)


def _loss_kernel(h_ref, w_ref, lab_ref, maskw_ref, out_ref,
                 m_run, s_run, lab_acc, loss_acc, cnt_acc):
    t = pl.program_id(0)
    v = pl.program_id(1)

    @pl.when(v == 0)
    def _init():
        m_run[...] = jnp.full((TBLK, 1), -jnp.inf, jnp.float32)
        s_run[...] = jnp.zeros((TBLK, 1), jnp.float32)
        lab_acc[...] = jnp.zeros((TBLK, 1), jnp.float32)

    logits = jnp.dot(h_ref[...].astype(jnp.bfloat16), w_ref[...],
                     preferred_element_type=jnp.float32)
    bm = jnp.max(logits, axis=1, keepdims=True)
    mnew = jnp.maximum(m_run[...], bm)
    s_run[...] = (s_run[...] * jnp.exp(m_run[...] - mnew)
                  + jnp.sum(jnp.exp(logits - mnew), axis=1, keepdims=True))
    m_run[...] = mnew

    rel = lab_ref[...] - v * VBLK
    hit = lax.broadcasted_iota(jnp.int32, (TBLK, VBLK), 1) == rel
    lab_acc[...] += jnp.sum(jnp.where(hit, logits, 0.0), axis=1, keepdims=True)

    @pl.when(v == NVB - 1)
    def _fin():
        tok_ll = lab_acc[...] - (jnp.log(s_run[...]) + m_run[...])

        @pl.when(t == 0)
        def _zero():
            loss_acc[0, 0] = 0.0
            cnt_acc[0, 0] = 0.0

        loss_acc[0, 0] += jnp.sum(maskw_ref[...] * tok_ll)
        cnt_acc[0, 0] += jnp.sum(maskw_ref[...])

        @pl.when(t == NTB - 1)
        def _out():
            out_ref[...] = jnp.full((1, 1), -loss_acc[0, 0] / cnt_acc[0, 0],
                                    jnp.float32)


def _masked_loss(h, w_bf, labels, maskw):
    return pl.pallas_call(
        _loss_kernel,
        grid=(NTB, NVB),
        in_specs=[
            pl.BlockSpec((TBLK, D), lambda t, v: (t, 0)),
            pl.BlockSpec((D, VBLK), lambda t, v: (0, v)),
            pl.BlockSpec((TBLK, 1), lambda t, v: (t, 0)),
            pl.BlockSpec((TBLK, 1), lambda t, v: (t, 0)),
        ],
        out_specs=pl.BlockSpec((1, 1), lambda t, v: (0, 0)),
        out_shape=jax.ShapeDtypeStruct((1, 1), jnp.float32),
        scratch_shapes=[
            pltpu.VMEM((TBLK, 1), jnp.float32),
            pltpu.VMEM((TBLK, 1), jnp.float32),
            pltpu.VMEM((TBLK, 1), jnp.float32),
            pltpu.SMEM((1, 1), jnp.float32),
            pltpu.SMEM((1, 1), jnp.float32),
        ],
    )(h, w_bf, labels, maskw)


def kernel(x, embed, W_out, rand_times, perm_rand, subset_rand1,
           subset_rand2, random_tokens):
    xr = x.reshape(B, SUB, LANE)
    pr = perm_rand.reshape(B, SUB, LANE)
    s1r = subset_rand1.reshape(B, SUB, LANE)
    s2r = subset_rand2.reshape(B, SUB, LANE)
    rtokr = random_tokens.reshape(B, SUB, LANE)
    rt2 = rand_times.reshape(B, 1, 1)

    mask_f, ids = _compute_masks(rt2, xr, pr, s1r, s2r, rtokr)
    h = _gather_rows(embed, ids.reshape(B * N))
    w_bf = W_out.astype(jnp.bfloat16)
    out = _masked_loss(h, w_bf, x.reshape(B * N, 1), mask_f.reshape(B * N, 1))
    return out[0, 0]


# speedup vs baseline: 2.0373x; 1.1508x over previous
import functools

import jax
import jax.numpy as jnp
import numpy as np
from jax import lax
from jax.experimental import pallas as pl
from jax.experimental.pallas import tpu as pltpu
from jax.experimental.pallas import tpu_sc as plsc

B, N, D, V = 2, 2048, 768, 8192
MASK_ID = V
SUB, LANE = 16, 128
NO_REPLACE_PROB = np.float32(0.15)
RANDOM_TOKEN_PROB_EFF = np.float32(0.05 * (1.0 - 0.15))

TBLK = 256
VBLK = 2048
NTB, NVB = (B * N) // TBLK, V // VBLK


def _bitonic_argsort(v, ilin, li, si):
    idx = ilin
    for klog in range(1, 12):
        kbit = 1 << klog
        asc = (ilin & kbit) == 0
        for jlog in reversed(range(klog)):
            d = 1 << jlog
            if d < LANE:
                ax, sh = 1, d
                lower = (li & d) == 0
            else:
                ax, sh = 0, d // LANE
                lower = (si & sh) == 0
            pv = jnp.where(lower, jnp.roll(v, -sh, axis=ax), jnp.roll(v, sh, axis=ax))
            pidx = jnp.where(lower, jnp.roll(idx, -sh, axis=ax), jnp.roll(idx, sh, axis=ax))
            gtp = (v > pv) | ((v == pv) & (idx > pidx))
            take = gtp == (asc == lower)
            v = jnp.where(take, pv, v)
            idx = jnp.where(take, pidx, idx)
    return idx


def _mask_kernel(rt_ref, x_ref, perm_ref, s1_ref, s2_ref, rtok_ref,
                 mask_ref, ids_ref):
    li = lax.broadcasted_iota(jnp.int32, (SUB, LANE), 1)
    si = lax.broadcasted_iota(jnp.int32, (SUB, LANE), 0)
    ilin = si * LANE + li

    sidx = _bitonic_argsort(perm_ref[0], ilin, li, si)
    rt = rt_ref[0, 0, 0]
    numtok = jnp.maximum((1.0 - rt) * np.float32(N), np.float32(1.0))
    mask = sidx.astype(jnp.float32) < numtok

    msum = jnp.sum(mask.astype(jnp.float32))
    p1 = np.float32(N) - msum
    nm1 = jnp.maximum(msum * NO_REPLACE_PROB, np.float32(0.0))
    a2 = jnp.where(mask, s1_ref[0], np.float32(-1.0))
    sidx2 = _bitonic_argsort(a2, ilin, li, si)
    norep = mask & ((sidx2.astype(jnp.float32) - p1) < nm1)
    rep = mask & (~norep)

    rsum = jnp.sum(rep.astype(jnp.float32))
    p2 = np.float32(N) - rsum
    nm2 = jnp.maximum(rsum * RANDOM_TOKEN_PROB_EFF, np.float32(0.0))
    a3 = jnp.where(rep, s2_ref[0], np.float32(-1.0))
    sidx3 = _bitonic_argsort(a3, ilin, li, si)
    rndm = rep & ((sidx3.astype(jnp.float32) - p2) < nm2)
    rep_final = rep & (~rndm)

    x2 = jnp.where(rndm, rtok_ref[0], x_ref[0])
    ids_ref[0] = jnp.where(rep_final, MASK_ID, x2)
    mask_ref[0] = mask.astype(jnp.float32)


def _compute_masks(rt2, xr, pr, s1r, s2r, rtokr):
    blk = pl.BlockSpec((1, SUB, LANE), lambda b: (b, 0, 0))
    return pl.pallas_call(
        _mask_kernel,
        grid=(B,),
        in_specs=[pl.BlockSpec((1, 1, 1), lambda b: (b, 0, 0), memory_space=pltpu.SMEM),
                  blk, blk, blk, blk, blk],
        out_specs=[blk, blk],
        out_shape=[
            jax.ShapeDtypeStruct((B, SUB, LANE), jnp.float32),
            jax.ShapeDtypeStruct((B, SUB, LANE), jnp.int32),
        ],
    )(rt2, xr, pr, s1r, s2r, rtokr)


_SC_CORES, _SC_SUBCORES = 2, 16
_NW = _SC_CORES * _SC_SUBCORES
_BPW = (B * N) // _NW


def _gather_body(table_hbm, idx_hbm, out_hbm, idx_v, rows_v, sem):
    wid = lax.axis_index("s") * _SC_CORES + lax.axis_index("c")
    base = wid * _BPW
    pltpu.sync_copy(idx_hbm.at[pl.ds(base, _BPW)], idx_v)
    pltpu.async_copy(table_hbm.at[idx_v], rows_v, sem).wait()
    pltpu.sync_copy(rows_v, out_hbm.at[pl.ds(base, _BPW)])


def _gather_rows(table_bf, ids_flat):
    mesh = plsc.VectorSubcoreMesh(core_axis_name="c", subcore_axis_name="s")
    k = pl.kernel(
        _gather_body,
        out_type=jax.ShapeDtypeStruct((B * N, D), jnp.float32),
        mesh=mesh,
        scratch_types=[
            pltpu.VMEM((_BPW,), jnp.int32),
            pltpu.VMEM((_BPW, D), jnp.float32),
            pltpu.SemaphoreType.DMA,
        ],
    )
    return k(table_bf, ids_flat)


def _loss_kernel(h_ref, w_ref, lab_ref, maskw_ref, out_ref,
                 loss_acc, cnt_acc):
    t = pl.program_id(0)
    logits = jnp.dot(h_ref[...].astype(jnp.bfloat16), w_ref[...],
                     preferred_element_type=jnp.float32)
    m = jnp.max(logits, axis=1, keepdims=True)
    s = jnp.sum(jnp.exp(logits - m), axis=1, keepdims=True)
    hit = lax.broadcasted_iota(jnp.int32, (TBLK, V), 1) == lab_ref[...]
    lab = jnp.sum(jnp.where(hit, logits, 0.0), axis=1, keepdims=True)
    tok_ll = lab - (jnp.log(s) + m)

    @pl.when(t == 0)
    def _zero():
        loss_acc[0, 0] = 0.0
        cnt_acc[0, 0] = 0.0

    loss_acc[0, 0] += jnp.sum(maskw_ref[...] * tok_ll)
    cnt_acc[0, 0] += jnp.sum(maskw_ref[...])

    @pl.when(t == NTB - 1)
    def _out():
        out_ref[...] = jnp.full((1, 1), -loss_acc[0, 0] / cnt_acc[0, 0],
                                jnp.float32)


def _masked_loss(h, w_bf, labels, maskw):
    return pl.pallas_call(
        _loss_kernel,
        grid=(NTB,),
        in_specs=[
            pl.BlockSpec((TBLK, D), lambda t: (t, 0)),
            pl.BlockSpec((D, V), lambda t: (0, 0)),
            pl.BlockSpec((TBLK, 1), lambda t: (t, 0)),
            pl.BlockSpec((TBLK, 1), lambda t: (t, 0)),
        ],
        out_specs=pl.BlockSpec((1, 1), lambda t: (0, 0)),
        out_shape=jax.ShapeDtypeStruct((1, 1), jnp.float32),
        scratch_shapes=[
            pltpu.SMEM((1, 1), jnp.float32),
            pltpu.SMEM((1, 1), jnp.float32),
        ],
    )(h, w_bf, labels, maskw)


def kernel(x, embed, W_out, rand_times, perm_rand, subset_rand1,
           subset_rand2, random_tokens):
    xr = x.reshape(B, SUB, LANE)
    pr = perm_rand.reshape(B, SUB, LANE)
    s1r = subset_rand1.reshape(B, SUB, LANE)
    s2r = subset_rand2.reshape(B, SUB, LANE)
    rtokr = random_tokens.reshape(B, SUB, LANE)
    rt2 = rand_times.reshape(B, 1, 1)

    mask_f, ids = _compute_masks(rt2, xr, pr, s1r, s2r, rtokr)
    h = _gather_rows(embed, ids.reshape(B * N))
    w_bf = W_out.astype(jnp.bfloat16)
    out = _masked_loss(h, w_bf, x.reshape(B * N, 1), mask_f.reshape(B * N, 1))
    return out[0, 0]
